# baseline (speedup 1.0000x reference)
import functools
import numpy as np
import jax
import jax.numpy as jnp
from jax.experimental import pallas as pl
from jax.experimental.pallas import tpu as pltpu

_D_MODEL = 1024
_N_HEADS = 16
_FACTOR = 5
_B = 2
_L = 2048
_DH = _D_MODEL // _N_HEADS
_U = min(_FACTOR * int(np.ceil(np.log(_L))), _L)
_SCALE = 1.0 / np.sqrt(_DH)
_NEG = np.float32(-np.inf)


def _tf_rotl(v, d):
    d = np.uint32(d)
    return ((v << d) | (v >> np.uint32(32 - d))).astype(np.uint32)


def _tf2x32(k1, k2, x1, x2):
    ks = [np.uint32(k1), np.uint32(k2),
          np.uint32(np.uint32(k1) ^ np.uint32(k2) ^ np.uint32(0x1BD11BDA))]
    rot = [np.array([13, 15, 26, 6]), np.array([17, 29, 16, 24])]
    x = [np.asarray(x1, np.uint32) + ks[0], np.asarray(x2, np.uint32) + ks[1]]
    for i in range(5):
        for r in rot[i % 2]:
            x[0] = (x[0] + x[1]).astype(np.uint32)
            x[1] = x[0] ^ _tf_rotl(x[1], r)
        x[0] = (x[0] + ks[(i + 1) % 3]).astype(np.uint32)
        x[1] = (x[1] + ks[(i + 2) % 3] + np.uint32(i + 1)).astype(np.uint32)
    return x[0], x[1]


def _tf_count(key, count):
    flat = np.asarray(count, np.uint32).ravel()
    h = flat.shape[0] // 2
    o0, o1 = _tf2x32(key[0], key[1], flat[:h], flat[h:])
    return np.concatenate([o0, o1]).reshape(np.shape(count))


def _tf_random_bits(key, shape):
    n = int(np.prod(shape))
    io = np.arange(n, dtype=np.uint64)
    b1, b2 = _tf2x32(key[0], key[1],
                     (io >> np.uint64(32)).astype(np.uint32).reshape(shape),
                     (io & np.uint64(0xFFFFFFFF)).astype(np.uint32).reshape(shape))
    return b1 ^ b2


def _sample_count_T():
    key = _tf_count(np.array([0, 0], np.uint32), np.array([0, 123], np.uint32))
    b1, b2 = _tf2x32(key[0], key[1], np.zeros(2, np.uint32),
                     np.arange(2, dtype=np.uint32))
    subkeys = np.stack([b1, b2], axis=1)
    higher = _tf_random_bits(subkeys[0], (_L, _U))
    lower = _tf_random_bits(subkeys[1], (_L, _U))
    span = np.uint32(_L)
    mult = np.uint32((((2 ** 16) % _L) ** 2) % _L)
    idx = (((higher % span) * mult + lower % span) % span).astype(np.int64)
    c = np.zeros((_L, _L), np.float32)
    np.add.at(c, (np.arange(_L)[:, None], idx), 1.0)
    return np.ascontiguousarray(c.T)


_CF_T = _sample_count_T()
_BB_T = np.where(_CF_T > 0, np.float32(0.0), np.float32(-np.inf))


_RBLK = 256
_NBLK = _L // _RBLK


def _proj_kernel(xq_ref, xk_ref, xv_ref, wq_ref, bq_ref, wk_ref, bk_ref,
                 wv_ref, bv_ref, qo_ref, ko_ref, vo_ref, co_ref, carry_ref):
    j = pl.program_id(1)
    qo_ref[0] = jnp.dot(xq_ref[0], wq_ref[...],
                        preferred_element_type=jnp.float32) + bq_ref[...]
    ko_ref[0] = jnp.dot(xk_ref[0], wk_ref[...],
                        preferred_element_type=jnp.float32) + bk_ref[...]
    vv = jnp.dot(xv_ref[0], wv_ref[...],
                 preferred_element_type=jnp.float32) + bv_ref[...]
    vo_ref[0] = vv
    carry = jnp.where(j == 0, jnp.float32(0.0), carry_ref[...])
    ri = jax.lax.broadcasted_iota(jnp.int32, (_RBLK, _RBLK), 0)
    ci = jax.lax.broadcasted_iota(jnp.int32, (_RBLK, _RBLK), 1)
    tri = (ri >= ci).astype(jnp.float32)
    co_ref[0] = jnp.dot(tri, vv, preferred_element_type=jnp.float32) + carry
    carry_ref[...] = carry + jnp.sum(vv, axis=0, keepdims=True)


def _run_proj(queries, keys, values, Wq, bq, Wk, bk, Wv, bv):
    x_spec = pl.BlockSpec((1, _RBLK, _D_MODEL), lambda b, j: (b, j, 0))
    w_spec = pl.BlockSpec((_D_MODEL, _D_MODEL), lambda b, j: (0, 0))
    b_spec = pl.BlockSpec((1, _D_MODEL), lambda b, j: (0, 0))
    o_spec = pl.BlockSpec((1, _RBLK, _D_MODEL), lambda b, j: (b, j, 0))
    shp = jax.ShapeDtypeStruct((_B, _L, _D_MODEL), jnp.float32)
    return pl.pallas_call(
        _proj_kernel,
        grid=(_B, _NBLK),
        in_specs=[x_spec, x_spec, x_spec,
                  w_spec, b_spec, w_spec, b_spec, w_spec, b_spec],
        out_specs=[o_spec, o_spec, o_spec, o_spec],
        out_shape=[shp, shp, shp, shp],
        scratch_shapes=[pltpu.VMEM((1, _D_MODEL), jnp.float32)],
    )(queries, keys, values, Wq, bq.reshape(1, -1), Wk, bk.reshape(1, -1),
      Wv, bv.reshape(1, -1))


_KBLK = 256
_NKBLK = _L // _KBLK


def _one_head(q, k, v, ctx, bb_ref, cf_ref, oh_ref):
    ones_row = jnp.ones((1, _KBLK), jnp.float32)
    smax = jnp.full((1, _L), _NEG, jnp.float32)
    ssum = jnp.zeros((1, _L), jnp.float32)
    for kb in range(_NKBLK):
        ksl = slice(kb * _KBLK, (kb + 1) * _KBLK)
        st = jax.lax.dot_general(k[ksl, :], q, (((1,), (1,)), ((), ())),
                                 preferred_element_type=jnp.float32)
        smax = jnp.maximum(
            smax, jnp.max(st + bb_ref[ksl, :], axis=0, keepdims=True))
        ssum = ssum + jnp.dot(ones_row, st * cf_ref[ksl, :],
                              preferred_element_type=jnp.float32)
    m = smax - ssum * np.float32(1.0 / _L)

    iota = jax.lax.broadcasted_iota(jnp.int32, (1, _L), 1)
    for u in range(_U):
        cur = jnp.max(m, axis=1, keepdims=True)
        idx = jnp.min(jnp.where(m == cur, iota, _L), axis=1, keepdims=True)
        row = iota == idx
        oh_ref[u:u + 1, :] = row.astype(jnp.float32)
        m = jnp.where(row, _NEG, m)

    oh = oh_ref[...]
    q_sel = jax.lax.dot_general(oh, q, (((1,), (0,)), ((), ())),
                                preferred_element_type=jnp.float32)
    iota_f = jax.lax.broadcasted_iota(jnp.int32, (_U, _L), 1).astype(jnp.float32)
    idx_val = jnp.sum(oh * iota_f, axis=1, keepdims=True)

    s = jax.lax.dot_general(q_sel, k, (((1,), (1,)), ((), ())),
                            preferred_element_type=jnp.float32)
    s = s * np.float32(_SCALE)
    s = jnp.where(iota_f > idx_val, _NEG, s)
    smx = jnp.max(s, axis=1, keepdims=True)
    p = jnp.exp(s - smx)
    attn = p / jnp.sum(p, axis=1, keepdims=True)

    upd = jnp.dot(attn, v, preferred_element_type=jnp.float32)
    scat = jax.lax.dot_general(oh, upd, (((0,), (0,)), ((), ())),
                               preferred_element_type=jnp.float32)
    sel = jax.lax.dot_general(oh, jnp.ones((_U, _DH), jnp.float32),
                              (((0,), (0,)), ((), ())),
                              preferred_element_type=jnp.float32)
    return jnp.where(sel > 0.5, scat, ctx)


def _attn_kernel(q_ref, k_ref, v_ref, bb_ref, cf_ref, ctx_ref, out_ref,
                 oh_ref):
    for hh in range(2):
        sl = slice(hh * _DH, (hh + 1) * _DH)
        out_ref[0, :, sl] = _one_head(q_ref[0][:, sl], k_ref[0][:, sl],
                                      v_ref[0][:, sl], ctx_ref[0][:, sl],
                                      bb_ref, cf_ref, oh_ref)


def _run_attn(qkvc, bb, cf):
    q, k, v, ctx = qkvc
    nh2 = _N_HEADS // 2
    h_spec = pl.BlockSpec((1, _L, 2 * _DH), lambda i: (i // nh2, 0, i % nh2))
    c_spec = pl.BlockSpec((_L, _L), lambda i: (0, 0))
    return pl.pallas_call(
        _attn_kernel,
        grid=(_B * nh2,),
        in_specs=[h_spec, h_spec, h_spec, c_spec, c_spec, h_spec],
        out_specs=h_spec,
        out_shape=jax.ShapeDtypeStruct((_B, _L, _D_MODEL), jnp.float32),
        scratch_shapes=[pltpu.VMEM((_U, _L), jnp.float32)],
    )(q, k, v, bb, cf, ctx)


def kernel(queries, keys, values, Wq, bq, Wk, bk, Wv, bv):
    bb = jnp.asarray(_BB_T)
    cf = jnp.asarray(_CF_T)
    q, k, v, ctx = _run_proj(queries, keys, values, Wq, bq, Wk, bk, Wv, bv)
    return _run_attn((q, k, v, ctx), bb, cf)


# speedup vs baseline: 2.1486x; 2.1486x over previous
import functools
import numpy as np
import jax
import jax.numpy as jnp
from jax.experimental import pallas as pl
from jax.experimental.pallas import tpu as pltpu

_D_MODEL = 1024
_N_HEADS = 16
_FACTOR = 5
_B = 2
_L = 2048
_DH = _D_MODEL // _N_HEADS
_U = min(_FACTOR * int(np.ceil(np.log(_L))), _L)
_SCALE = 1.0 / np.sqrt(_DH)
_NEG = np.float32(-np.inf)


def _tf_rotl(v, d):
    d = np.uint32(d)
    return ((v << d) | (v >> np.uint32(32 - d))).astype(np.uint32)


def _tf2x32(k1, k2, x1, x2):
    ks = [np.uint32(k1), np.uint32(k2),
          np.uint32(np.uint32(k1) ^ np.uint32(k2) ^ np.uint32(0x1BD11BDA))]
    rot = [np.array([13, 15, 26, 6]), np.array([17, 29, 16, 24])]
    x = [np.asarray(x1, np.uint32) + ks[0], np.asarray(x2, np.uint32) + ks[1]]
    for i in range(5):
        for r in rot[i % 2]:
            x[0] = (x[0] + x[1]).astype(np.uint32)
            x[1] = x[0] ^ _tf_rotl(x[1], r)
        x[0] = (x[0] + ks[(i + 1) % 3]).astype(np.uint32)
        x[1] = (x[1] + ks[(i + 2) % 3] + np.uint32(i + 1)).astype(np.uint32)
    return x[0], x[1]


def _tf_count(key, count):
    flat = np.asarray(count, np.uint32).ravel()
    h = flat.shape[0] // 2
    o0, o1 = _tf2x32(key[0], key[1], flat[:h], flat[h:])
    return np.concatenate([o0, o1]).reshape(np.shape(count))


def _tf_random_bits(key, shape):
    n = int(np.prod(shape))
    io = np.arange(n, dtype=np.uint64)
    b1, b2 = _tf2x32(key[0], key[1],
                     (io >> np.uint64(32)).astype(np.uint32).reshape(shape),
                     (io & np.uint64(0xFFFFFFFF)).astype(np.uint32).reshape(shape))
    return b1 ^ b2


def _sample_count_T():
    key = _tf_count(np.array([0, 0], np.uint32), np.array([0, 123], np.uint32))
    b1, b2 = _tf2x32(key[0], key[1], np.zeros(2, np.uint32),
                     np.arange(2, dtype=np.uint32))
    subkeys = np.stack([b1, b2], axis=1)
    higher = _tf_random_bits(subkeys[0], (_L, _U))
    lower = _tf_random_bits(subkeys[1], (_L, _U))
    span = np.uint32(_L)
    mult = np.uint32((((2 ** 16) % _L) ** 2) % _L)
    idx = (((higher % span) * mult + lower % span) % span).astype(np.int64)
    c = np.zeros((_L, _L), np.float32)
    np.add.at(c, (np.arange(_L)[:, None], idx), 1.0)
    return np.ascontiguousarray(c.T)


_CF_T = _sample_count_T()
_BB_T = np.where(_CF_T > 0, np.float32(0.0), np.float32(-np.inf))


_RBLK = 256
_NBLK = _L // _RBLK


def _proj_kernel(xq_ref, xk_ref, xv_ref, wq_ref, bq_ref, wk_ref, bk_ref,
                 wv_ref, bv_ref, qo_ref, ko_ref, vo_ref, co_ref, carry_ref):
    j = pl.program_id(1)
    qo_ref[0] = jnp.dot(xq_ref[0], wq_ref[...],
                        preferred_element_type=jnp.float32) + bq_ref[...]
    ko_ref[0] = jnp.dot(xk_ref[0], wk_ref[...],
                        preferred_element_type=jnp.float32) + bk_ref[...]
    vv = jnp.dot(xv_ref[0], wv_ref[...],
                 preferred_element_type=jnp.float32) + bv_ref[...]
    vo_ref[0] = vv
    carry = jnp.where(j == 0, jnp.float32(0.0), carry_ref[...])
    ri = jax.lax.broadcasted_iota(jnp.int32, (_RBLK, _RBLK), 0)
    ci = jax.lax.broadcasted_iota(jnp.int32, (_RBLK, _RBLK), 1)
    tri = (ri >= ci).astype(jnp.float32)
    co_ref[0] = jnp.dot(tri, vv, preferred_element_type=jnp.float32) + carry
    carry_ref[...] = carry + jnp.sum(vv, axis=0, keepdims=True)


def _run_proj(queries, keys, values, Wq, bq, Wk, bk, Wv, bv):
    x_spec = pl.BlockSpec((1, _RBLK, _D_MODEL), lambda b, j: (b, j, 0))
    w_spec = pl.BlockSpec((_D_MODEL, _D_MODEL), lambda b, j: (0, 0))
    b_spec = pl.BlockSpec((1, _D_MODEL), lambda b, j: (0, 0))
    o_spec = pl.BlockSpec((1, _RBLK, _D_MODEL), lambda b, j: (b, j, 0))
    shp = jax.ShapeDtypeStruct((_B, _L, _D_MODEL), jnp.float32)
    return pl.pallas_call(
        _proj_kernel,
        grid=(_B, _NBLK),
        in_specs=[x_spec, x_spec, x_spec,
                  w_spec, b_spec, w_spec, b_spec, w_spec, b_spec],
        out_specs=[o_spec, o_spec, o_spec, o_spec],
        out_shape=[shp, shp, shp, shp],
        scratch_shapes=[pltpu.VMEM((1, _D_MODEL), jnp.float32)],
    )(queries, keys, values, Wq, bq.reshape(1, -1), Wk, bk.reshape(1, -1),
      Wv, bv.reshape(1, -1))


_KBLK = 256
_NKBLK = _L // _KBLK


def _measure_kernel(q_ref, k_ref, bb_ref, cf_ref, m_ref):
    ones_row = jnp.ones((1, _KBLK), jnp.float32)
    for hh in range(2):
        sl = slice(hh * _DH, (hh + 1) * _DH)
        q = q_ref[0][:, sl]
        k = k_ref[0][:, sl]
        smax = jnp.full((1, _L), _NEG, jnp.float32)
        ssum = jnp.zeros((1, _L), jnp.float32)
        for kb in range(_NKBLK):
            ksl = slice(kb * _KBLK, (kb + 1) * _KBLK)
            st = jax.lax.dot_general(k[ksl, :], q, (((1,), (1,)), ((), ())),
                                     preferred_element_type=jnp.float32)
            smax = jnp.maximum(
                smax, jnp.max(st + bb_ref[ksl, :], axis=0, keepdims=True))
            ssum = ssum + jnp.dot(ones_row, st * cf_ref[ksl, :],
                                  preferred_element_type=jnp.float32)
        m_ref[:, hh, :] = smax - ssum * np.float32(1.0 / _L)


def _topk_kernel(m_ref, oh_ref):
    m = m_ref[...]
    nh = _B * _N_HEADS
    iota = jax.lax.broadcasted_iota(jnp.int32, (nh, _L), 1)
    for u in range(_U):
        cur = jnp.max(m, axis=1, keepdims=True)
        idx = jnp.min(jnp.where(m == cur, iota, _L), axis=1, keepdims=True)
        rows = iota == idx
        oh_ref[:, u, :] = rows.astype(jnp.float32)
        m = jnp.where(rows, _NEG, m)


def _attn_kernel(q_ref, k_ref, v_ref, ctx_ref, oh_ref, out_ref):
    for hh in range(2):
        sl = slice(hh * _DH, (hh + 1) * _DH)
        q = q_ref[0][:, sl]
        k = k_ref[0][:, sl]
        v = v_ref[0][:, sl]
        ctx = ctx_ref[0][:, sl]
        oh = oh_ref[hh]

        q_sel = jax.lax.dot_general(oh, q, (((1,), (0,)), ((), ())),
                                    preferred_element_type=jnp.float32)
        iota_f = jax.lax.broadcasted_iota(
            jnp.int32, (_U, _L), 1).astype(jnp.float32)
        idx_val = jnp.sum(oh * iota_f, axis=1, keepdims=True)

        s = jax.lax.dot_general(q_sel, k, (((1,), (1,)), ((), ())),
                                preferred_element_type=jnp.float32)
        s = s * np.float32(_SCALE)
        s = jnp.where(iota_f > idx_val, _NEG, s)
        smx = jnp.max(s, axis=1, keepdims=True)
        p = jnp.exp(s - smx)
        attn = p / jnp.sum(p, axis=1, keepdims=True)

        upd = jnp.dot(attn, v, preferred_element_type=jnp.float32)
        scat = jax.lax.dot_general(oh, upd, (((0,), (0,)), ((), ())),
                                   preferred_element_type=jnp.float32)
        sel = jax.lax.dot_general(oh, jnp.ones((_U, _DH), jnp.float32),
                                  (((0,), (0,)), ((), ())),
                                  preferred_element_type=jnp.float32)
        out_ref[0, :, sl] = jnp.where(sel > 0.5, scat, ctx)


def kernel(queries, keys, values, Wq, bq, Wk, bk, Wv, bv):
    bb = jnp.asarray(_BB_T)
    cf = jnp.asarray(_CF_T)
    nh = _B * _N_HEADS
    nh2 = nh // 2
    q, k, v, ctx = _run_proj(queries, keys, values, Wq, bq, Wk, bk, Wv, bv)

    h_spec = pl.BlockSpec((1, _L, 2 * _DH),
                          lambda i: (i // (_N_HEADS // 2), 0,
                                     i % (_N_HEADS // 2)))
    c_spec = pl.BlockSpec((_L, _L), lambda i: (0, 0))
    m = pl.pallas_call(
        _measure_kernel,
        grid=(nh2,),
        in_specs=[h_spec, h_spec, c_spec, c_spec],
        out_specs=pl.BlockSpec((1, 2, _L), lambda i: (i, 0, 0)),
        out_shape=jax.ShapeDtypeStruct((nh2, 2, _L), jnp.float32),
    )(q, k, bb, cf)

    oh = pl.pallas_call(
        _topk_kernel,
        grid=(1,),
        in_specs=[pl.BlockSpec((nh, _L), lambda i: (0, 0))],
        out_specs=pl.BlockSpec((nh, _U, _L), lambda i: (0, 0, 0)),
        out_shape=jax.ShapeDtypeStruct((nh, _U, _L), jnp.float32),
    )(m.reshape(nh, _L))

    return pl.pallas_call(
        _attn_kernel,
        grid=(nh2,),
        in_specs=[h_spec, h_spec, h_spec, h_spec,
                  pl.BlockSpec((2, _U, _L), lambda i: (i, 0, 0))],
        out_specs=h_spec,
        out_shape=jax.ShapeDtypeStruct((_B, _L, _D_MODEL), jnp.float32),
    )(q, k, v, ctx, oh)


# speedup vs baseline: 2.1977x; 1.0229x over previous
import functools
import numpy as np
import jax
import jax.numpy as jnp
from jax.experimental import pallas as pl
from jax.experimental.pallas import tpu as pltpu

_D_MODEL = 1024
_N_HEADS = 16
_FACTOR = 5
_B = 2
_L = 2048
_DH = _D_MODEL // _N_HEADS
_U = min(_FACTOR * int(np.ceil(np.log(_L))), _L)
_SCALE = 1.0 / np.sqrt(_DH)
_NEG = np.float32(-np.inf)


def _tf_rotl(v, d):
    d = np.uint32(d)
    return ((v << d) | (v >> np.uint32(32 - d))).astype(np.uint32)


def _tf2x32(k1, k2, x1, x2):
    ks = [np.uint32(k1), np.uint32(k2),
          np.uint32(np.uint32(k1) ^ np.uint32(k2) ^ np.uint32(0x1BD11BDA))]
    rot = [np.array([13, 15, 26, 6]), np.array([17, 29, 16, 24])]
    x = [np.asarray(x1, np.uint32) + ks[0], np.asarray(x2, np.uint32) + ks[1]]
    for i in range(5):
        for r in rot[i % 2]:
            x[0] = (x[0] + x[1]).astype(np.uint32)
            x[1] = x[0] ^ _tf_rotl(x[1], r)
        x[0] = (x[0] + ks[(i + 1) % 3]).astype(np.uint32)
        x[1] = (x[1] + ks[(i + 2) % 3] + np.uint32(i + 1)).astype(np.uint32)
    return x[0], x[1]


def _tf_count(key, count):
    flat = np.asarray(count, np.uint32).ravel()
    h = flat.shape[0] // 2
    o0, o1 = _tf2x32(key[0], key[1], flat[:h], flat[h:])
    return np.concatenate([o0, o1]).reshape(np.shape(count))


def _tf_random_bits(key, shape):
    n = int(np.prod(shape))
    io = np.arange(n, dtype=np.uint64)
    b1, b2 = _tf2x32(key[0], key[1],
                     (io >> np.uint64(32)).astype(np.uint32).reshape(shape),
                     (io & np.uint64(0xFFFFFFFF)).astype(np.uint32).reshape(shape))
    return b1 ^ b2


def _sample_count_T():
    key = _tf_count(np.array([0, 0], np.uint32), np.array([0, 123], np.uint32))
    b1, b2 = _tf2x32(key[0], key[1], np.zeros(2, np.uint32),
                     np.arange(2, dtype=np.uint32))
    subkeys = np.stack([b1, b2], axis=1)
    higher = _tf_random_bits(subkeys[0], (_L, _U))
    lower = _tf_random_bits(subkeys[1], (_L, _U))
    span = np.uint32(_L)
    mult = np.uint32((((2 ** 16) % _L) ** 2) % _L)
    idx = (((higher % span) * mult + lower % span) % span).astype(np.int64)
    c = np.zeros((_L, _L), np.float32)
    np.add.at(c, (np.arange(_L)[:, None], idx), 1.0)
    return np.ascontiguousarray(c.T)


_CF_T = _sample_count_T()
_BB_T = np.where(_CF_T > 0, np.float32(0.0), np.float32(-np.inf))


_RBLK = 256
_NBLK = _L // _RBLK


def _proj_kernel(xq_ref, xk_ref, xv_ref, wq_ref, bq_ref, wk_ref, bk_ref,
                 wv_ref, bv_ref, qo_ref, ko_ref, vo_ref, co_ref, carry_ref):
    j = pl.program_id(1)
    qo_ref[0] = jnp.dot(xq_ref[0], wq_ref[...],
                        preferred_element_type=jnp.float32) + bq_ref[...]
    ko_ref[0] = jnp.dot(xk_ref[0], wk_ref[...],
                        preferred_element_type=jnp.float32) + bk_ref[...]
    vv = jnp.dot(xv_ref[0], wv_ref[...],
                 preferred_element_type=jnp.float32) + bv_ref[...]
    vo_ref[0] = vv
    carry = jnp.where(j == 0, jnp.float32(0.0), carry_ref[...])
    ri = jax.lax.broadcasted_iota(jnp.int32, (_RBLK, _RBLK), 0)
    ci = jax.lax.broadcasted_iota(jnp.int32, (_RBLK, _RBLK), 1)
    tri = (ri >= ci).astype(jnp.float32)
    co_ref[0] = jnp.dot(tri, vv, preferred_element_type=jnp.float32) + carry
    carry_ref[...] = carry + jnp.sum(vv, axis=0, keepdims=True)


def _run_proj(queries, keys, values, Wq, bq, Wk, bk, Wv, bv):
    x_spec = pl.BlockSpec((1, _RBLK, _D_MODEL), lambda b, j: (b, j, 0))
    w_spec = pl.BlockSpec((_D_MODEL, _D_MODEL), lambda b, j: (0, 0))
    b_spec = pl.BlockSpec((1, _D_MODEL), lambda b, j: (0, 0))
    o_spec = pl.BlockSpec((1, _RBLK, _D_MODEL), lambda b, j: (b, j, 0))
    shp = jax.ShapeDtypeStruct((_B, _L, _D_MODEL), jnp.float32)
    return pl.pallas_call(
        _proj_kernel,
        grid=(_B, _NBLK),
        in_specs=[x_spec, x_spec, x_spec,
                  w_spec, b_spec, w_spec, b_spec, w_spec, b_spec],
        out_specs=[o_spec, o_spec, o_spec, o_spec],
        out_shape=[shp, shp, shp, shp],
        scratch_shapes=[pltpu.VMEM((1, _D_MODEL), jnp.float32)],
    )(queries, keys, values, Wq, bq.reshape(1, -1), Wk, bk.reshape(1, -1),
      Wv, bv.reshape(1, -1))


_KBLK = 256
_NKBLK = _L // _KBLK


def _measure_kernel(q_ref, k_ref, bb_ref, cf_ref, m_ref):
    ones_row = jnp.ones((1, _KBLK), jnp.float32)
    for hh in range(2):
        sl = slice(hh * _DH, (hh + 1) * _DH)
        q = q_ref[0][:, sl]
        k = k_ref[0][:, sl]
        smax = jnp.full((1, _L), _NEG, jnp.float32)
        ssum = jnp.zeros((1, _L), jnp.float32)
        for kb in range(_NKBLK):
            ksl = slice(kb * _KBLK, (kb + 1) * _KBLK)
            st = jax.lax.dot_general(k[ksl, :], q, (((1,), (1,)), ((), ())),
                                     preferred_element_type=jnp.float32)
            smax = jnp.maximum(
                smax, jnp.max(st + bb_ref[ksl, :], axis=0, keepdims=True))
            ssum = ssum + jnp.dot(ones_row, st * cf_ref[ksl, :],
                                  preferred_element_type=jnp.float32)
        m_ref[:, hh, :] = smax - ssum * np.float32(1.0 / _L)


def _attn_kernel(q_ref, k_ref, v_ref, ctx_ref, m_ref, out_ref, oh_ref):
    i = pl.program_id(0)

    @pl.when(i == 0)
    def _topk():
        m = m_ref[...]
        nh = _B * _N_HEADS
        iota = jax.lax.broadcasted_iota(jnp.int32, (nh, _L), 1)
        for u in range(_U):
            cur = jnp.max(m, axis=1, keepdims=True)
            idx = jnp.min(jnp.where(m == cur, iota, _L), axis=1,
                          keepdims=True)
            rows = iota == idx
            oh_ref[:, u, :] = rows.astype(jnp.float32)
            m = jnp.where(rows, _NEG, m)

    for hh in range(2):
        sl = slice(hh * _DH, (hh + 1) * _DH)
        q = q_ref[0][:, sl]
        k = k_ref[0][:, sl]
        v = v_ref[0][:, sl]
        ctx = ctx_ref[0][:, sl]
        oh = oh_ref[2 * i + hh]

        q_sel = jax.lax.dot_general(oh, q, (((1,), (0,)), ((), ())),
                                    preferred_element_type=jnp.float32)
        iota_f = jax.lax.broadcasted_iota(
            jnp.int32, (_U, _L), 1).astype(jnp.float32)
        idx_val = jnp.sum(oh * iota_f, axis=1, keepdims=True)

        s = jax.lax.dot_general(q_sel, k, (((1,), (1,)), ((), ())),
                                preferred_element_type=jnp.float32)
        s = s * np.float32(_SCALE)
        s = jnp.where(iota_f > idx_val, _NEG, s)
        smx = jnp.max(s, axis=1, keepdims=True)
        p = jnp.exp(s - smx)
        attn = p / jnp.sum(p, axis=1, keepdims=True)

        upd = jnp.dot(attn, v, preferred_element_type=jnp.float32)
        scat = jax.lax.dot_general(oh, upd, (((0,), (0,)), ((), ())),
                                   preferred_element_type=jnp.float32)
        sel = jax.lax.dot_general(oh, jnp.ones((_U, _DH), jnp.float32),
                                  (((0,), (0,)), ((), ())),
                                  preferred_element_type=jnp.float32)
        out_ref[0, :, sl] = jnp.where(sel > 0.5, scat, ctx)


def kernel(queries, keys, values, Wq, bq, Wk, bk, Wv, bv):
    bb = jnp.asarray(_BB_T)
    cf = jnp.asarray(_CF_T)
    nh = _B * _N_HEADS
    nh2 = nh // 2
    q, k, v, ctx = _run_proj(queries, keys, values, Wq, bq, Wk, bk, Wv, bv)

    h_spec = pl.BlockSpec((1, _L, 2 * _DH),
                          lambda i: (i // (_N_HEADS // 2), 0,
                                     i % (_N_HEADS // 2)))
    c_spec = pl.BlockSpec((_L, _L), lambda i: (0, 0))
    m = pl.pallas_call(
        _measure_kernel,
        grid=(nh2,),
        in_specs=[h_spec, h_spec, c_spec, c_spec],
        out_specs=pl.BlockSpec((1, 2, _L), lambda i: (i, 0, 0)),
        out_shape=jax.ShapeDtypeStruct((nh2, 2, _L), jnp.float32),
    )(q, k, bb, cf)

    return pl.pallas_call(
        _attn_kernel,
        grid=(nh2,),
        in_specs=[h_spec, h_spec, h_spec, h_spec,
                  pl.BlockSpec((nh, _L), lambda i: (0, 0))],
        out_specs=h_spec,
        out_shape=jax.ShapeDtypeStruct((_B, _L, _D_MODEL), jnp.float32),
        scratch_shapes=[pltpu.VMEM((nh, _U, _L), jnp.float32)],
    )(q, k, v, ctx, m.reshape(nh, _L))


# speedup vs baseline: 2.2349x; 1.0169x over previous
import functools
import numpy as np
import jax
import jax.numpy as jnp
from jax.experimental import pallas as pl
from jax.experimental.pallas import tpu as pltpu

_D_MODEL = 1024
_N_HEADS = 16
_FACTOR = 5
_B = 2
_L = 2048
_DH = _D_MODEL // _N_HEADS
_U = min(_FACTOR * int(np.ceil(np.log(_L))), _L)
_SCALE = 1.0 / np.sqrt(_DH)
_NEG = np.float32(-np.inf)


def _tf_rotl(v, d):
    d = np.uint32(d)
    return ((v << d) | (v >> np.uint32(32 - d))).astype(np.uint32)


def _tf2x32(k1, k2, x1, x2):
    ks = [np.uint32(k1), np.uint32(k2),
          np.uint32(np.uint32(k1) ^ np.uint32(k2) ^ np.uint32(0x1BD11BDA))]
    rot = [np.array([13, 15, 26, 6]), np.array([17, 29, 16, 24])]
    x = [np.asarray(x1, np.uint32) + ks[0], np.asarray(x2, np.uint32) + ks[1]]
    for i in range(5):
        for r in rot[i % 2]:
            x[0] = (x[0] + x[1]).astype(np.uint32)
            x[1] = x[0] ^ _tf_rotl(x[1], r)
        x[0] = (x[0] + ks[(i + 1) % 3]).astype(np.uint32)
        x[1] = (x[1] + ks[(i + 2) % 3] + np.uint32(i + 1)).astype(np.uint32)
    return x[0], x[1]


def _tf_count(key, count):
    flat = np.asarray(count, np.uint32).ravel()
    h = flat.shape[0] // 2
    o0, o1 = _tf2x32(key[0], key[1], flat[:h], flat[h:])
    return np.concatenate([o0, o1]).reshape(np.shape(count))


def _tf_random_bits(key, shape):
    n = int(np.prod(shape))
    io = np.arange(n, dtype=np.uint64)
    b1, b2 = _tf2x32(key[0], key[1],
                     (io >> np.uint64(32)).astype(np.uint32).reshape(shape),
                     (io & np.uint64(0xFFFFFFFF)).astype(np.uint32).reshape(shape))
    return b1 ^ b2


def _sample_count_T():
    key = _tf_count(np.array([0, 0], np.uint32), np.array([0, 123], np.uint32))
    b1, b2 = _tf2x32(key[0], key[1], np.zeros(2, np.uint32),
                     np.arange(2, dtype=np.uint32))
    subkeys = np.stack([b1, b2], axis=1)
    higher = _tf_random_bits(subkeys[0], (_L, _U))
    lower = _tf_random_bits(subkeys[1], (_L, _U))
    span = np.uint32(_L)
    mult = np.uint32((((2 ** 16) % _L) ** 2) % _L)
    idx = (((higher % span) * mult + lower % span) % span).astype(np.int64)
    c = np.zeros((_L, _L), np.float32)
    np.add.at(c, (np.arange(_L)[:, None], idx), 1.0)
    return np.ascontiguousarray(c.T)


_CF_T = _sample_count_T()
_BB_T = np.where(_CF_T > 0, np.float32(0.0), np.float32(-np.inf))


_RBLK = 512
_NBLK = _L // _RBLK


def _proj_kernel(xq_ref, xk_ref, xv_ref, wq_ref, bq_ref, wk_ref, bk_ref,
                 wv_ref, bv_ref, qo_ref, ko_ref, vo_ref, co_ref, carry_ref):
    j = pl.program_id(1)
    qo_ref[0] = jnp.dot(xq_ref[0], wq_ref[...],
                        preferred_element_type=jnp.float32) + bq_ref[...]
    ko_ref[0] = jnp.dot(xk_ref[0], wk_ref[...],
                        preferred_element_type=jnp.float32) + bk_ref[...]
    vv = jnp.dot(xv_ref[0], wv_ref[...],
                 preferred_element_type=jnp.float32) + bv_ref[...]
    vo_ref[0] = vv
    carry = jnp.where(j == 0, jnp.float32(0.0), carry_ref[...])
    ri = jax.lax.broadcasted_iota(jnp.int32, (_RBLK, _RBLK), 0)
    ci = jax.lax.broadcasted_iota(jnp.int32, (_RBLK, _RBLK), 1)
    tri = (ri >= ci).astype(jnp.float32)
    co_ref[0] = jnp.dot(tri, vv, preferred_element_type=jnp.float32) + carry
    carry_ref[...] = carry + jnp.sum(vv, axis=0, keepdims=True)


def _run_proj(queries, keys, values, Wq, bq, Wk, bk, Wv, bv):
    x_spec = pl.BlockSpec((1, _RBLK, _D_MODEL), lambda b, j: (b, j, 0))
    w_spec = pl.BlockSpec((_D_MODEL, _D_MODEL), lambda b, j: (0, 0))
    b_spec = pl.BlockSpec((1, _D_MODEL), lambda b, j: (0, 0))
    o_spec = pl.BlockSpec((1, _RBLK, _D_MODEL), lambda b, j: (b, j, 0))
    shp = jax.ShapeDtypeStruct((_B, _L, _D_MODEL), jnp.float32)
    return pl.pallas_call(
        _proj_kernel,
        grid=(_B, _NBLK),
        in_specs=[x_spec, x_spec, x_spec,
                  w_spec, b_spec, w_spec, b_spec, w_spec, b_spec],
        out_specs=[o_spec, o_spec, o_spec, o_spec],
        out_shape=[shp, shp, shp, shp],
        scratch_shapes=[pltpu.VMEM((1, _D_MODEL), jnp.float32)],
    )(queries, keys, values, Wq, bq.reshape(1, -1), Wk, bk.reshape(1, -1),
      Wv, bv.reshape(1, -1))


_KBLK = 256
_NKBLK = _L // _KBLK


def _measure_kernel(q_ref, k_ref, bb_ref, cf_ref, m_ref):
    ones_row = jnp.ones((1, _KBLK), jnp.float32)
    for hh in range(2):
        sl = slice(hh * _DH, (hh + 1) * _DH)
        q = q_ref[0][:, sl]
        k = k_ref[0][:, sl]
        smax = jnp.full((1, _L), _NEG, jnp.float32)
        ssum = jnp.zeros((1, _L), jnp.float32)
        for kb in range(_NKBLK):
            ksl = slice(kb * _KBLK, (kb + 1) * _KBLK)
            st = jax.lax.dot_general(k[ksl, :], q, (((1,), (1,)), ((), ())),
                                     preferred_element_type=jnp.float32)
            smax = jnp.maximum(
                smax, jnp.max(st + bb_ref[ksl, :], axis=0, keepdims=True))
            ssum = ssum + jnp.dot(ones_row, st * cf_ref[ksl, :],
                                  preferred_element_type=jnp.float32)
        m_ref[:, hh, :] = smax - ssum * np.float32(1.0 / _L)


def _attn_kernel(q_ref, k_ref, v_ref, ctx_ref, m_ref, out_ref, oh_ref):
    i = pl.program_id(0)

    @pl.when(i == 0)
    def _topk():
        m = m_ref[...]
        nh = _B * _N_HEADS
        iota = jax.lax.broadcasted_iota(jnp.int32, (nh, _L), 1)
        for u in range(_U):
            cur = jnp.max(m, axis=1, keepdims=True)
            idx = jnp.min(jnp.where(m == cur, iota, _L), axis=1,
                          keepdims=True)
            rows = iota == idx
            oh_ref[:, u, :] = rows.astype(jnp.float32)
            m = jnp.where(rows, _NEG, m)

    for hh in range(2):
        sl = slice(hh * _DH, (hh + 1) * _DH)
        q = q_ref[0][:, sl]
        k = k_ref[0][:, sl]
        v = v_ref[0][:, sl]
        ctx = ctx_ref[0][:, sl]
        oh = oh_ref[2 * i + hh]

        q_sel = jax.lax.dot_general(oh, q, (((1,), (0,)), ((), ())),
                                    preferred_element_type=jnp.float32)
        iota_f = jax.lax.broadcasted_iota(
            jnp.int32, (_U, _L), 1).astype(jnp.float32)
        idx_val = jnp.sum(oh * iota_f, axis=1, keepdims=True)

        s = jax.lax.dot_general(q_sel, k, (((1,), (1,)), ((), ())),
                                preferred_element_type=jnp.float32)
        s = s * np.float32(_SCALE)
        s = jnp.where(iota_f > idx_val, _NEG, s)
        smx = jnp.max(s, axis=1, keepdims=True)
        p = jnp.exp(s - smx)
        attn = p / jnp.sum(p, axis=1, keepdims=True)

        upd = jnp.dot(attn, v, preferred_element_type=jnp.float32)
        scat = jax.lax.dot_general(oh, upd, (((0,), (0,)), ((), ())),
                                   preferred_element_type=jnp.float32)
        sel = jax.lax.dot_general(oh, jnp.ones((_U, _DH), jnp.float32),
                                  (((0,), (0,)), ((), ())),
                                  preferred_element_type=jnp.float32)
        out_ref[0, :, sl] = jnp.where(sel > 0.5, scat, ctx)


def kernel(queries, keys, values, Wq, bq, Wk, bk, Wv, bv):
    bb = jnp.asarray(_BB_T)
    cf = jnp.asarray(_CF_T)
    nh = _B * _N_HEADS
    nh2 = nh // 2
    q, k, v, ctx = _run_proj(queries, keys, values, Wq, bq, Wk, bk, Wv, bv)

    h_spec = pl.BlockSpec((1, _L, 2 * _DH),
                          lambda i: (i // (_N_HEADS // 2), 0,
                                     i % (_N_HEADS // 2)))
    c_spec = pl.BlockSpec((_L, _L), lambda i: (0, 0))
    m = pl.pallas_call(
        _measure_kernel,
        grid=(nh2,),
        in_specs=[h_spec, h_spec, c_spec, c_spec],
        out_specs=pl.BlockSpec((1, 2, _L), lambda i: (i, 0, 0)),
        out_shape=jax.ShapeDtypeStruct((nh2, 2, _L), jnp.float32),
    )(q, k, bb, cf)

    return pl.pallas_call(
        _attn_kernel,
        grid=(nh2,),
        in_specs=[h_spec, h_spec, h_spec, h_spec,
                  pl.BlockSpec((nh, _L), lambda i: (0, 0))],
        out_specs=h_spec,
        out_shape=jax.ShapeDtypeStruct((_B, _L, _D_MODEL), jnp.float32),
        scratch_shapes=[pltpu.VMEM((nh, _U, _L), jnp.float32)],
    )(q, k, v, ctx, m.reshape(nh, _L))
